# Initial kernel scaffold; baseline (speedup 1.0000x reference)
import jax
import jax.numpy as jnp
from jax.experimental import pallas as pl


def kernel(x, edge_index, W1, b1, W2, b2, W3, b3):
    raise NotImplementedError("write your pallas kernel here")


# speedup vs baseline: 6.9788x; 6.9788x over previous
import functools
import jax
import jax.numpy as jnp
from jax import lax
from jax.experimental import pallas as pl
from jax.experimental.pallas import tpu as pltpu
from jax.experimental.pallas import tpu_sc as plsc

N = 10000
D = 128
E = 320000
NC, NS = 2, 16
NW = NC * NS
N_PAD = 10240
CHUNK = 128
STEPS = 80
HALF = STEPS // 2
E_PAD = NW * STEPS * CHUNK
ROWS_PER_TILE = N_PAD // NS
NBLK = N_PAD // 128

_mesh = plsc.VectorSubcoreMesh(
    core_axis_name="c", subcore_axis_name="s", num_cores=NC, num_subcores=NS)


_deg_scratch = [
    pltpu.VMEM((STEPS, CHUNK), jnp.int32),
    pltpu.VMEM((CHUNK, D), jnp.float32),
    pltpu.VMEM_SHARED((N_PAD, D), jnp.float32),
    pltpu.SemaphoreType.DMA,
]


def _deg_body(idx_hbm, ones_hbm, zeros_hbm, out_hbm, dst_v, ones_v, acc_sh, sem):
    cid = lax.axis_index("c")
    sid = lax.axis_index("s")
    tid = cid * NS + sid
    pltpu.sync_copy(idx_hbm.at[tid, 1], dst_v)
    pltpu.sync_copy(ones_hbm, ones_v)
    pltpu.sync_copy(zeros_hbm.at[pl.ds(sid * ROWS_PER_TILE, ROWS_PER_TILE)],
                    acc_sh.at[pl.ds(sid * ROWS_PER_TILE, ROWS_PER_TILE)])
    plsc.subcore_barrier()

    def body(j, carry):
        pltpu.async_copy(ones_v, acc_sh.at[dst_v.at[j]], sem, add=True)
        return carry
    lax.fori_loop(0, STEPS, body, 0)

    def drain(j, carry):
        pltpu.make_async_copy(ones_v, acc_sh.at[dst_v.at[0]], sem).wait()
        return carry
    lax.fori_loop(0, STEPS, drain, 0)
    plsc.subcore_barrier()
    pltpu.sync_copy(acc_sh.at[pl.ds(sid * ROWS_PER_TILE, ROWS_PER_TILE)],
                    out_hbm.at[cid].at[pl.ds(sid * ROWS_PER_TILE, ROWS_PER_TILE)])


_deg_kernel = pl.kernel(
    _deg_body,
    out_type=jax.ShapeDtypeStruct((NC, N_PAD, D), jnp.float32),
    mesh=_mesh,
    scratch_types=_deg_scratch,
)


_scatter_scratch = [
    pltpu.VMEM((2, HALF, CHUNK), jnp.int32),
    pltpu.VMEM((CHUNK, D), jnp.float32),
    pltpu.VMEM((CHUNK, D), jnp.float32),
    pltpu.VMEM_SHARED((N_PAD, D), jnp.float32),
    pltpu.SemaphoreType.DMA,
    pltpu.SemaphoreType.DMA,
    pltpu.SemaphoreType.DMA,
]


def _scatter_body(g_hbm, idx_hbm, zeros_hbm, out_hbm,
                  idx_v, r0, r1, acc_sh, gsem, s0, s1):
    cid = lax.axis_index("c")
    sid = lax.axis_index("s")
    tid = cid * NS + sid
    pltpu.sync_copy(zeros_hbm.at[pl.ds(sid * ROWS_PER_TILE, ROWS_PER_TILE)],
                    acc_sh.at[pl.ds(sid * ROWS_PER_TILE, ROWS_PER_TILE)])
    plsc.subcore_barrier()

    def half_pass(h, carry):
        pltpu.sync_copy(idx_hbm.at[tid, :, pl.ds(h * HALF, HALF)], idx_v)

        def body(i, carry2):
            j0 = 2 * i
            j1 = 2 * i + 1
            pltpu.async_copy(g_hbm.at[idx_v.at[0, j0]], r0, gsem).wait()
            pltpu.async_copy(r0, acc_sh.at[idx_v.at[1, j0]], s0, add=True)
            pltpu.async_copy(g_hbm.at[idx_v.at[0, j1]], r1, gsem).wait()
            pltpu.async_copy(r1, acc_sh.at[idx_v.at[1, j1]], s1, add=True)
            pltpu.make_async_copy(r0, acc_sh.at[idx_v.at[1, j0]], s0).wait()
            pltpu.make_async_copy(r1, acc_sh.at[idx_v.at[1, j1]], s1).wait()
            return carry2
        lax.fori_loop(0, HALF // 2, body, 0)
        return carry
    lax.fori_loop(0, 2, half_pass, 0)
    plsc.subcore_barrier()
    pltpu.sync_copy(acc_sh.at[pl.ds(sid * ROWS_PER_TILE, ROWS_PER_TILE)],
                    out_hbm.at[cid].at[pl.ds(sid * ROWS_PER_TILE, ROWS_PER_TILE)])


_scatter_kernel = pl.kernel(
    _scatter_body,
    out_type=jax.ShapeDtypeStruct((NC, N_PAD, D), jnp.float32),
    mesh=_mesh,
    scratch_types=_scatter_scratch,
)


def _dinv_body(deg_ref, o_ref):
    d = deg_ref[0, :, 0:1] + deg_ref[1, :, 0:1] + 1.0
    o_ref[...] = lax.rsqrt(d)


_dinv_kernel = pl.pallas_call(
    _dinv_body,
    grid=(NBLK,),
    in_specs=[pl.BlockSpec((2, 128, D), lambda i: (0, i, 0))],
    out_specs=pl.BlockSpec((128, 1), lambda i: (i, 0)),
    out_shape=jax.ShapeDtypeStruct((N_PAD, 1), jnp.float32),
)


def _mm_scale_body(x_ref, w_ref, dinv_ref, o_ref):
    h = jnp.dot(x_ref[...], w_ref[...], preferred_element_type=jnp.float32)
    o_ref[...] = h * dinv_ref[...]


_mm_scale = pl.pallas_call(
    _mm_scale_body,
    grid=(NBLK,),
    in_specs=[
        pl.BlockSpec((128, D), lambda i: (i, 0)),
        pl.BlockSpec((D, D), lambda i: (0, 0)),
        pl.BlockSpec((128, 1), lambda i: (i, 0)),
    ],
    out_specs=pl.BlockSpec((128, D), lambda i: (i, 0)),
    out_shape=jax.ShapeDtypeStruct((N_PAD, D), jnp.float32),
)


def _combine_mm_body(p_ref, g_ref, dinv_ref, b_ref, w_ref, o_ref):
    s = p_ref[0] + p_ref[1] + g_ref[...]
    a = jnp.maximum(dinv_ref[...] * s + b_ref[...], 0.0)
    h = jnp.dot(a, w_ref[...], preferred_element_type=jnp.float32)
    o_ref[...] = h * dinv_ref[...]


_combine_mm = pl.pallas_call(
    _combine_mm_body,
    grid=(NBLK,),
    in_specs=[
        pl.BlockSpec((2, 128, D), lambda i: (0, i, 0)),
        pl.BlockSpec((128, D), lambda i: (i, 0)),
        pl.BlockSpec((128, 1), lambda i: (i, 0)),
        pl.BlockSpec((1, D), lambda i: (0, 0)),
        pl.BlockSpec((D, D), lambda i: (0, 0)),
    ],
    out_specs=pl.BlockSpec((128, D), lambda i: (i, 0)),
    out_shape=jax.ShapeDtypeStruct((N_PAD, D), jnp.float32),
)


def _final_body(p_ref, g_ref, dinv_ref, b_ref, o_ref):
    s = p_ref[0] + p_ref[1] + g_ref[...]
    o_ref[...] = jnp.maximum(dinv_ref[...] * s + b_ref[...], 0.0)


_final_kernel = pl.pallas_call(
    _final_body,
    grid=(NBLK,),
    in_specs=[
        pl.BlockSpec((2, 128, D), lambda i: (0, i, 0)),
        pl.BlockSpec((128, D), lambda i: (i, 0)),
        pl.BlockSpec((128, 1), lambda i: (i, 0)),
        pl.BlockSpec((1, D), lambda i: (0, 0)),
    ],
    out_specs=pl.BlockSpec((128, D), lambda i: (i, 0)),
    out_shape=jax.ShapeDtypeStruct((N_PAD, D), jnp.float32),
)


@jax.jit
def _run(x, edge_index, W1, b1, W2, b2, W3, b3):
    src = edge_index[0].astype(jnp.int32)
    dst = edge_index[1].astype(jnp.int32)
    src_p = jnp.concatenate(
        [src, jnp.zeros((E_PAD - E,), jnp.int32)]).reshape(NW, 1, STEPS, CHUNK)
    dst_p = jnp.concatenate(
        [dst, jnp.full((E_PAD - E,), N, jnp.int32)]).reshape(NW, 1, STEPS, CHUNK)
    idx_p = jnp.concatenate([src_p, dst_p], axis=1)
    x_p = jnp.pad(x, ((0, N_PAD - N), (0, 0)))
    zeros_deg = jnp.zeros((N_PAD, D), jnp.float32)
    ones_deg = jnp.ones((CHUNK, D), jnp.float32)
    zeros_big = jnp.zeros((N_PAD, D), jnp.float32)
    b1r = b1.reshape(1, D)
    b2r = b2.reshape(1, D)
    b3r = b3.reshape(1, D)

    degp = _deg_kernel(idx_p, ones_deg, zeros_deg)
    dinv = _dinv_kernel(degp)
    g = _mm_scale(x_p, W1, dinv)
    p = _scatter_kernel(g, idx_p, zeros_big)
    g = _combine_mm(p, g, dinv, b1r, W2)
    p = _scatter_kernel(g, idx_p, zeros_big)
    g = _combine_mm(p, g, dinv, b2r, W3)
    p = _scatter_kernel(g, idx_p, zeros_big)
    out = _final_kernel(p, g, dinv, b3r)
    return out[:N]


def kernel(x, edge_index, W1, b1, W2, b2, W3, b3):
    return _run(x, edge_index, W1, b1, W2, b2, W3, b3)


# speedup vs baseline: 8.4433x; 1.2098x over previous
import functools
import jax
import jax.numpy as jnp
from jax import lax
from jax.experimental import pallas as pl
from jax.experimental.pallas import tpu as pltpu
from jax.experimental.pallas import tpu_sc as plsc

N = 10000
D = 128
E = 320000
NC, NS = 2, 16
NW = NC * NS
N_PAD = 10240
CHUNK = 128
CHUNKS_TOTAL = 2560
E_PAD = CHUNKS_TOTAL * CHUNK
C0 = 140
C1 = 20
SSIZE = 20
IDX_PAD = CHUNKS_TOTAL + SSIZE
STEPS = 80
ROWS_PER_TILE = N_PAD // NS
NBLK = N_PAD // 128

_mesh = plsc.VectorSubcoreMesh(
    core_axis_name="c", subcore_axis_name="s", num_cores=NC, num_subcores=NS)


_deg_scratch = [
    pltpu.VMEM((STEPS, CHUNK), jnp.int32),
    pltpu.VMEM((CHUNK, D), jnp.float32),
    pltpu.VMEM_SHARED((N_PAD, D), jnp.float32),
    pltpu.SemaphoreType.DMA,
]


def _deg_body(idx_hbm, ones_hbm, zeros_hbm, out_hbm, dst_v, ones_v, acc_sh, sem):
    cid = lax.axis_index("c")
    sid = lax.axis_index("s")
    tid = cid * NS + sid
    pltpu.sync_copy(idx_hbm.at[pl.ds(tid * STEPS, STEPS), 1], dst_v)
    pltpu.sync_copy(ones_hbm, ones_v)
    pltpu.sync_copy(zeros_hbm.at[pl.ds(sid * ROWS_PER_TILE, ROWS_PER_TILE)],
                    acc_sh.at[pl.ds(sid * ROWS_PER_TILE, ROWS_PER_TILE)])
    plsc.subcore_barrier()

    def body(j, carry):
        pltpu.async_copy(ones_v, acc_sh.at[dst_v.at[j]], sem, add=True)
        return carry
    lax.fori_loop(0, STEPS, body, 0)

    def drain(j, carry):
        pltpu.make_async_copy(ones_v, acc_sh.at[dst_v.at[0]], sem).wait()
        return carry
    lax.fori_loop(0, STEPS, drain, 0)
    plsc.subcore_barrier()
    pltpu.sync_copy(acc_sh.at[pl.ds(sid * ROWS_PER_TILE, ROWS_PER_TILE)],
                    out_hbm.at[cid].at[pl.ds(sid * ROWS_PER_TILE, ROWS_PER_TILE)])


_deg_kernel = pl.kernel(
    _deg_body,
    out_type=jax.ShapeDtypeStruct((NC, N_PAD, D), jnp.float32),
    mesh=_mesh,
    scratch_types=_deg_scratch,
)


_scatter_scratch = [
    pltpu.VMEM((2, SSIZE, 2, CHUNK), jnp.int32),
    pltpu.VMEM((CHUNK, D), jnp.float32),
    pltpu.VMEM((CHUNK, D), jnp.float32),
    pltpu.VMEM_SHARED((N_PAD, D), jnp.float32),
    pltpu.SemaphoreType.DMA,
    pltpu.SemaphoreType.DMA,
    pltpu.SemaphoreType.DMA,
    pltpu.SemaphoreType.DMA,
    pltpu.SemaphoreType.DMA,
]


def _scatter_body(g_hbm, idx_hbm, zeros_hbm, out_hbm,
                  idx_v, r0, r1, acc_sh, g0, g1, s0, s1, isem):
    cid = lax.axis_index("c")
    sid = lax.axis_index("s")
    nchunk = jnp.where(cid == 0, C0, C1)
    base = jnp.where(cid == 0, sid * C0, NS * C0 + sid * C1)
    pltpu.sync_copy(zeros_hbm.at[pl.ds(sid * ROWS_PER_TILE, ROWS_PER_TILE)],
                    acc_sh.at[pl.ds(sid * ROWS_PER_TILE, ROWS_PER_TILE)])
    pltpu.sync_copy(idx_hbm.at[pl.ds(base, SSIZE)], idx_v.at[0])
    pltpu.async_copy(idx_hbm.at[pl.ds(base + SSIZE, SSIZE)], idx_v.at[1], isem)
    plsc.subcore_barrier()

    pltpu.async_copy(g_hbm.at[idx_v.at[0, 0, 0]], r0, g0)
    pltpu.async_copy(g_hbm.at[idx_v.at[0, 1, 0]], r1, g1)
    pltpu.make_async_copy(g_hbm.at[idx_v.at[0, 0, 0]], r0, g0).wait()
    pltpu.async_copy(r0, acc_sh.at[idx_v.at[0, 0, 1]], s0, add=True)
    pltpu.make_async_copy(g_hbm.at[idx_v.at[0, 1, 0]], r1, g1).wait()
    pltpu.async_copy(r1, acc_sh.at[idx_v.at[0, 1, 1]], s1, add=True)

    def body(i, carry):
        j0 = 2 * i
        blk = lax.rem(lax.div(j0, SSIZE), 2)
        off = lax.rem(j0, SSIZE)

        @pl.when(off == 0)
        def _new_block():
            pltpu.make_async_copy(
                idx_hbm.at[pl.ds(base, SSIZE)], idx_v.at[0], isem).wait()
            pltpu.async_copy(
                idx_hbm.at[pl.ds(base + j0 + SSIZE, SSIZE)],
                idx_v.at[1 - blk], isem)

        pltpu.make_async_copy(r0, acc_sh.at[idx_v.at[0, 0, 1]], s0).wait()
        pltpu.async_copy(g_hbm.at[idx_v.at[blk, off, 0]], r0, g0)
        pltpu.make_async_copy(r1, acc_sh.at[idx_v.at[0, 0, 1]], s1).wait()
        pltpu.async_copy(g_hbm.at[idx_v.at[blk, off + 1, 0]], r1, g1)
        pltpu.make_async_copy(g_hbm.at[idx_v.at[blk, off, 0]], r0, g0).wait()
        pltpu.async_copy(r0, acc_sh.at[idx_v.at[blk, off, 1]], s0, add=True)
        pltpu.make_async_copy(g_hbm.at[idx_v.at[blk, off + 1, 0]], r1, g1).wait()
        pltpu.async_copy(r1, acc_sh.at[idx_v.at[blk, off + 1, 1]], s1, add=True)
        return carry
    lax.fori_loop(1, nchunk // 2, body, 0)
    pltpu.make_async_copy(r0, acc_sh.at[idx_v.at[0, 0, 1]], s0).wait()
    pltpu.make_async_copy(r1, acc_sh.at[idx_v.at[0, 0, 1]], s1).wait()
    pltpu.make_async_copy(idx_hbm.at[pl.ds(base, SSIZE)], idx_v.at[0], isem).wait()
    plsc.subcore_barrier()
    pltpu.sync_copy(acc_sh.at[pl.ds(sid * ROWS_PER_TILE, ROWS_PER_TILE)],
                    out_hbm.at[cid].at[pl.ds(sid * ROWS_PER_TILE, ROWS_PER_TILE)])


_scatter_kernel = pl.kernel(
    _scatter_body,
    out_type=jax.ShapeDtypeStruct((NC, N_PAD, D), jnp.float32),
    mesh=_mesh,
    scratch_types=_scatter_scratch,
)


def _dinv_scale_body(deg_ref, h_ref, dinv_ref, g_ref):
    d = deg_ref[0, :, 0:1] + deg_ref[1, :, 0:1] + 1.0
    dinv = lax.rsqrt(d)
    dinv_ref[...] = dinv
    g_ref[...] = h_ref[...] * dinv


_dinv_scale = pl.pallas_call(
    _dinv_scale_body,
    grid=(NBLK,),
    in_specs=[
        pl.BlockSpec((2, 128, D), lambda i: (0, i, 0)),
        pl.BlockSpec((128, D), lambda i: (i, 0)),
    ],
    out_specs=[
        pl.BlockSpec((128, 1), lambda i: (i, 0)),
        pl.BlockSpec((128, D), lambda i: (i, 0)),
    ],
    out_shape=[
        jax.ShapeDtypeStruct((N_PAD, 1), jnp.float32),
        jax.ShapeDtypeStruct((N_PAD, D), jnp.float32),
    ],
)


def _mm_body(x_ref, w_ref, o_ref):
    o_ref[...] = jnp.dot(x_ref[...], w_ref[...],
                         preferred_element_type=jnp.float32)


_mm_kernel = pl.pallas_call(
    _mm_body,
    grid=(NBLK,),
    in_specs=[
        pl.BlockSpec((128, D), lambda i: (i, 0)),
        pl.BlockSpec((D, D), lambda i: (0, 0)),
    ],
    out_specs=pl.BlockSpec((128, D), lambda i: (i, 0)),
    out_shape=jax.ShapeDtypeStruct((N_PAD, D), jnp.float32),
)


def _combine_mm_body(p_ref, g_ref, dinv_ref, b_ref, w_ref, o_ref):
    s = p_ref[0] + p_ref[1] + g_ref[...]
    a = jnp.maximum(dinv_ref[...] * s + b_ref[...], 0.0)
    h = jnp.dot(a, w_ref[...], preferred_element_type=jnp.float32)
    o_ref[...] = h * dinv_ref[...]


_combine_mm = pl.pallas_call(
    _combine_mm_body,
    grid=(NBLK,),
    in_specs=[
        pl.BlockSpec((2, 128, D), lambda i: (0, i, 0)),
        pl.BlockSpec((128, D), lambda i: (i, 0)),
        pl.BlockSpec((128, 1), lambda i: (i, 0)),
        pl.BlockSpec((1, D), lambda i: (0, 0)),
        pl.BlockSpec((D, D), lambda i: (0, 0)),
    ],
    out_specs=pl.BlockSpec((128, D), lambda i: (i, 0)),
    out_shape=jax.ShapeDtypeStruct((N_PAD, D), jnp.float32),
)


def _final_body(p_ref, g_ref, dinv_ref, b_ref, o_ref):
    s = p_ref[0] + p_ref[1] + g_ref[...]
    o_ref[...] = jnp.maximum(dinv_ref[...] * s + b_ref[...], 0.0)


_final_kernel = pl.pallas_call(
    _final_body,
    grid=(NBLK,),
    in_specs=[
        pl.BlockSpec((2, 128, D), lambda i: (0, i, 0)),
        pl.BlockSpec((128, D), lambda i: (i, 0)),
        pl.BlockSpec((128, 1), lambda i: (i, 0)),
        pl.BlockSpec((1, D), lambda i: (0, 0)),
    ],
    out_specs=pl.BlockSpec((128, D), lambda i: (i, 0)),
    out_shape=jax.ShapeDtypeStruct((N_PAD, D), jnp.float32),
)


@jax.jit
def _run(x, edge_index, W1, b1, W2, b2, W3, b3):
    src = edge_index[0].astype(jnp.int32)
    dst = edge_index[1].astype(jnp.int32)
    src_p = jnp.concatenate(
        [src, jnp.zeros((E_PAD - E,), jnp.int32)]).reshape(CHUNKS_TOTAL, 1, CHUNK)
    dst_p = jnp.concatenate(
        [dst, jnp.full((E_PAD - E,), N, jnp.int32)]).reshape(CHUNKS_TOTAL, 1, CHUNK)
    idx_p = jnp.concatenate([src_p, dst_p], axis=1)
    idx_p = jnp.pad(idx_p, ((0, IDX_PAD - CHUNKS_TOTAL), (0, 0), (0, 0)))
    x_p = jnp.pad(x, ((0, N_PAD - N), (0, 0)))
    zeros_deg = jnp.zeros((N_PAD, D), jnp.float32)
    ones_deg = jnp.ones((CHUNK, D), jnp.float32)
    zeros_big = jnp.zeros((N_PAD, D), jnp.float32)
    b1r = b1.reshape(1, D)
    b2r = b2.reshape(1, D)
    b3r = b3.reshape(1, D)

    degp = _deg_kernel(idx_p, ones_deg, zeros_deg)
    h1 = _mm_kernel(x_p, W1)
    dinv, g = _dinv_scale(degp, h1)
    p = _scatter_kernel(g, idx_p, zeros_big)
    g = _combine_mm(p, g, dinv, b1r, W2)
    p = _scatter_kernel(g, idx_p, zeros_big)
    g = _combine_mm(p, g, dinv, b2r, W3)
    p = _scatter_kernel(g, idx_p, zeros_big)
    out = _final_kernel(p, g, dinv, b3r)
    return out[:N]


def kernel(x, edge_index, W1, b1, W2, b2, W3, b3):
    return _run(x, edge_index, W1, b1, W2, b2, W3, b3)
